# Initial kernel scaffold; baseline (speedup 1.0000x reference)
import jax
import jax.numpy as jnp
from jax.experimental import pallas as pl


def kernel(front, back, mask, we, be, wf, bf, wg, bg, wh, bh):
    raise NotImplementedError("write your pallas kernel here")


# speedup vs baseline: 2.3615x; 2.3615x over previous
import jax
import jax.numpy as jnp
from jax.experimental import pallas as pl
from jax.experimental.pallas import tpu as pltpu

EPS = 1e-8
HB = 16
LB = HB * 128
HALO = 512
R = LB // HALO


def _colsum7(x):
    acc = x[:, 128:128 + LB]
    for k in range(1, 7):
        acc = acc + x[:, 128 * (1 + k):128 * (1 + k) + LB]
    return acc


def _rowsum7(y):
    lane = jax.lax.broadcasted_iota(jnp.int32, (1, LB), 1) % 128
    acc = y
    zcol = jnp.zeros((y.shape[0], 3), y.dtype)
    for s in (1, 2, 3):
        zs = zcol[:, :s]
        ls = jnp.concatenate([y[:, s:], zs], axis=1)
        rs = jnp.concatenate([zs, y[:, :-s]], axis=1)
        acc = acc + jnp.where(lane < 128 - s, ls, 0.0)
        acc = acc + jnp.where(lane >= s, rs, 0.0)
    return acc


def _box_stats(asm):
    s = _rowsum7(_colsum7(asm))
    s2 = _rowsum7(_colsum7(asm * asm))
    mean = s * (1.0 / 49.0)
    var = jnp.maximum((s2 - 49.0 * mean * mean) * (1.0 / 48.0), 0.0)
    return mean, var


def _assemble(center, top, bot, i, nt):
    t = jnp.where(i == 0, 0.0, top[0])
    b = jnp.where(i == nt - 1, 0.0, bot[0])
    return jnp.concatenate([t, center[0], b], axis=1)


def _s_body(fc, ft, fb, bc, bt, bb, wf_r, bf_r, wg_r, bg_r,
            s_ref, mn_ref, mx_ref):
    i = pl.program_id(1)
    nt = pl.num_programs(1)
    fasm = _assemble(fc, ft, fb, i, nt)
    basm = _assemble(bc, bt, bb, i, nt)
    fm, fv = _box_stats(fasm)
    bm, bv = _box_stats(basm)
    fn = (fc[0] - fm) / (jnp.sqrt(fv) + EPS)
    bn = (bc[0] - bm) / (jnp.sqrt(bv) + EPS)
    FF = jnp.dot(wf_r[...], fn, preferred_element_type=jnp.float32) + bf_r[...]
    GG = jnp.dot(wg_r[...], bn, preferred_element_type=jnp.float32) + bg_r[...]
    d = jnp.sum(FF * GG, axis=0, keepdims=True)
    nf = jnp.sqrt(jnp.sum(FF * FF, axis=0, keepdims=True))
    ng = jnp.sqrt(jnp.sum(GG * GG, axis=0, keepdims=True))
    S = d / (nf * ng)
    s_ref[0] = S
    mn = jnp.min(S)
    mx = jnp.max(S)

    @pl.when(i == 0)
    def _():
        mn_ref[...] = jnp.broadcast_to(mn, (1, 1, 128))
        mx_ref[...] = jnp.broadcast_to(mx, (1, 1, 128))

    @pl.when(i != 0)
    def _():
        mn_ref[...] = jnp.minimum(mn_ref[...], mn)
        mx_ref[...] = jnp.maximum(mx_ref[...], mx)


def _fuse_body(fc, ft, fb, bc, bt, bb, s_in, we_r, be_r, wh_r, bh_r,
               smin_r, sinv_r, o_ref):
    b = pl.program_id(0)
    i = pl.program_id(1)
    nt = pl.num_programs(1)
    fasm = _assemble(fc, ft, fb, i, nt)
    basm = _assemble(bc, bt, bb, i, nt)
    fm, fv = _box_stats(fasm)
    bm, bv = _box_stats(basm)
    fn = (fc[0] - fm) / (jnp.sqrt(fv) + EPS)
    ad = fn * jnp.sqrt(bv) + bm
    EE = jnp.dot(we_r[...], ad, preferred_element_type=jnp.float32) + be_r[...]
    HH = jnp.dot(wh_r[...], bc[0], preferred_element_type=jnp.float32) + bh_r[...]
    sn = (s_in[0] - smin_r[b]) * sinv_r[b]
    o_ref[0] = sn * EE + (1.0 - sn) * HH


def _specs(B, C, HW):
    nblk = HW // HALO
    center = pl.BlockSpec((1, C, LB), lambda b, i: (b, 0, i))
    top = pl.BlockSpec((1, C, HALO),
                       lambda b, i: (b, 0, jnp.maximum(R * i - 1, 0)))
    bot = pl.BlockSpec((1, C, HALO),
                       lambda b, i: (b, 0, jnp.minimum(R * i + R, nblk - 1)))
    w = pl.BlockSpec((C, C), lambda b, i: (0, 0))
    bias = pl.BlockSpec((C, 1), lambda b, i: (0, 0))
    return center, top, bot, w, bias


def kernel(front, back, mask, we, be, wf, bf, wg, bg, wh, bh):
    del mask
    B, C, H, W = front.shape
    HW = H * W
    T = H // HB
    f2 = front.reshape(B, C, HW)
    b2 = back.reshape(B, C, HW)
    center, top, bot, wspec, bspec = _specs(B, C, HW)
    params = pltpu.CompilerParams(
        dimension_semantics=("parallel", "arbitrary"),
        vmem_limit_bytes=56 * 1024 * 1024,
    )

    S, mn, mx = pl.pallas_call(
        _s_body,
        grid=(B, T),
        in_specs=[center, top, bot, center, top, bot,
                  wspec, bspec, wspec, bspec],
        out_specs=[
            pl.BlockSpec((1, 1, LB), lambda b, i: (b, 0, i)),
            pl.BlockSpec((1, 1, 128), lambda b, i: (b, 0, 0)),
            pl.BlockSpec((1, 1, 128), lambda b, i: (b, 0, 0)),
        ],
        out_shape=[
            jax.ShapeDtypeStruct((B, 1, HW), jnp.float32),
            jax.ShapeDtypeStruct((B, 1, 128), jnp.float32),
            jax.ShapeDtypeStruct((B, 1, 128), jnp.float32),
        ],
        compiler_params=params,
        name="psf_similarity",
    )(f2, f2, f2, b2, b2, b2,
      wf, bf.reshape(C, 1), wg, bg.reshape(C, 1))

    smin = mn[:, 0, 0]
    smax = mx[:, 0, 0]
    sinv = 1.0 / (smax - smin)

    fused = pl.pallas_call(
        _fuse_body,
        grid=(B, T),
        in_specs=[center, top, bot, center, top, bot,
                  pl.BlockSpec((1, 1, LB), lambda b, i: (b, 0, i)),
                  wspec, bspec, wspec, bspec,
                  pl.BlockSpec(memory_space=pltpu.SMEM),
                  pl.BlockSpec(memory_space=pltpu.SMEM)],
        out_specs=pl.BlockSpec((1, C, LB), lambda b, i: (b, 0, i)),
        out_shape=jax.ShapeDtypeStruct((B, C, HW), jnp.float32),
        compiler_params=params,
        name="psf_fuse",
    )(f2, f2, f2, b2, b2, b2, S,
      we, be.reshape(C, 1), wh, bh.reshape(C, 1), smin, sinv)

    return fused.reshape(B, C, H, W)


# speedup vs baseline: 3.6705x; 1.5543x over previous
import jax
import jax.numpy as jnp
from jax.experimental import pallas as pl
from jax.experimental.pallas import tpu as pltpu

EPS = 1e-8
HB = 16
LB = HB * 128
HALO = 512
R = LB // HALO


def _colsum7(x):
    base = HALO - 3 * 128
    acc = x[:, base:base + LB]
    for k in range(1, 7):
        acc = acc + x[:, base + 128 * k:base + 128 * k + LB]
    return acc


def _rowsum7(y):
    lane = jax.lax.broadcasted_iota(jnp.int32, (1, LB), 1) % 128
    acc = y
    zcol = jnp.zeros((y.shape[0], 3), y.dtype)
    for s in (1, 2, 3):
        zs = zcol[:, :s]
        ls = jnp.concatenate([y[:, s:], zs], axis=1)
        rs = jnp.concatenate([zs, y[:, :-s]], axis=1)
        acc = acc + jnp.where(lane < 128 - s, ls, 0.0)
        acc = acc + jnp.where(lane >= s, rs, 0.0)
    return acc


def _box_stats(asm):
    s = _rowsum7(_colsum7(asm))
    s2 = _rowsum7(_colsum7(asm * asm))
    mean = s * (1.0 / 49.0)
    var = jnp.maximum((s2 - 49.0 * mean * mean) * (1.0 / 48.0), 0.0)
    return mean, var


def _assemble(center, top, bot, i, nt):
    t = jnp.where(i == 0, 0.0, top[0])
    b = jnp.where(i == nt - 1, 0.0, bot[0])
    return jnp.concatenate([t, center[0], b], axis=1)


def _main_body(fc, ft, fb, bc, bt, bb,
               we_r, be_r, wf_r, bf_r, wg_r, bg_r, wh_r, bh_r,
               s_ref, mn_ref, mx_ref, d_ref, hh_ref):
    i = pl.program_id(1)
    nt = pl.num_programs(1)

    basm = _assemble(bc, bt, bb, i, nt)
    bm, bv = _box_stats(basm)
    bstd = jnp.sqrt(bv)
    bn = (bc[0] - bm) / (bstd + EPS)
    GG = jnp.dot(wg_r[...], bn, preferred_element_type=jnp.float32) + bg_r[...]

    fasm = _assemble(fc, ft, fb, i, nt)
    fm, fv = _box_stats(fasm)
    fn = (fc[0] - fm) / (jnp.sqrt(fv) + EPS)
    FF = jnp.dot(wf_r[...], fn, preferred_element_type=jnp.float32) + bf_r[...]

    d = jnp.sum(FF * GG, axis=0, keepdims=True)
    nf = jnp.sqrt(jnp.sum(FF * FF, axis=0, keepdims=True))
    ng = jnp.sqrt(jnp.sum(GG * GG, axis=0, keepdims=True))
    S = d / (nf * ng)
    s_ref[0] = S
    mn = jnp.min(S)
    mx = jnp.max(S)

    @pl.when(i == 0)
    def _():
        mn_ref[...] = jnp.broadcast_to(mn, (1, 1, 128))
        mx_ref[...] = jnp.broadcast_to(mx, (1, 1, 128))

    @pl.when(i != 0)
    def _():
        mn_ref[...] = jnp.minimum(mn_ref[...], mn)
        mx_ref[...] = jnp.maximum(mx_ref[...], mx)

    ad = fn * bstd + bm
    EE = jnp.dot(we_r[...], ad, preferred_element_type=jnp.float32) + be_r[...]
    HH = jnp.dot(wh_r[...], bc[0], preferred_element_type=jnp.float32) + bh_r[...]
    hh_ref[0] = HH
    d_ref[0] = EE - HH


def _blend_body(s_in, d_in, hh_in, smin_r, sinv_r, o_ref):
    b = pl.program_id(0)
    sn = (s_in[0] - smin_r[b]) * sinv_r[b]
    o_ref[0] = hh_in[0] + sn * d_in[0]


def kernel(front, back, mask, we, be, wf, bf, wg, bg, wh, bh):
    del mask
    B, C, H, W = front.shape
    HW = H * W
    T = H // HB
    nblk = HW // HALO
    f2 = front.reshape(B, C, HW)
    b2 = back.reshape(B, C, HW)

    center = pl.BlockSpec((1, C, LB), lambda b, i: (b, 0, i))
    top = pl.BlockSpec((1, C, HALO),
                       lambda b, i: (b, 0, jnp.maximum(R * i - 1, 0)))
    bot = pl.BlockSpec((1, C, HALO),
                       lambda b, i: (b, 0, jnp.minimum(R * i + R, nblk - 1)))
    wspec = pl.BlockSpec((C, C), lambda b, i: (0, 0))
    bspec = pl.BlockSpec((C, 1), lambda b, i: (0, 0))
    srow = pl.BlockSpec((1, 1, LB), lambda b, i: (b, 0, i))
    params = pltpu.CompilerParams(
        dimension_semantics=("parallel", "arbitrary"),
        vmem_limit_bytes=56 * 1024 * 1024,
    )

    S, mn, mx, D, HH = pl.pallas_call(
        _main_body,
        grid=(B, T),
        in_specs=[center, top, bot, center, top, bot,
                  wspec, bspec, wspec, bspec, wspec, bspec, wspec, bspec],
        out_specs=[
            srow,
            pl.BlockSpec((1, 1, 128), lambda b, i: (b, 0, 0)),
            pl.BlockSpec((1, 1, 128), lambda b, i: (b, 0, 0)),
            center,
            center,
        ],
        out_shape=[
            jax.ShapeDtypeStruct((B, 1, HW), jnp.float32),
            jax.ShapeDtypeStruct((B, 1, 128), jnp.float32),
            jax.ShapeDtypeStruct((B, 1, 128), jnp.float32),
            jax.ShapeDtypeStruct((B, C, HW), jnp.float32),
            jax.ShapeDtypeStruct((B, C, HW), jnp.float32),
        ],
        compiler_params=params,
        name="psf_main",
    )(f2, f2, f2, b2, b2, b2,
      we, be.reshape(C, 1), wf, bf.reshape(C, 1),
      wg, bg.reshape(C, 1), wh, bh.reshape(C, 1))

    smin = mn[:, 0, 0]
    smax = mx[:, 0, 0]
    sinv = 1.0 / (smax - smin)

    fused = pl.pallas_call(
        _blend_body,
        grid=(B, T),
        in_specs=[srow, center, center,
                  pl.BlockSpec(memory_space=pltpu.SMEM),
                  pl.BlockSpec(memory_space=pltpu.SMEM)],
        out_specs=center,
        out_shape=jax.ShapeDtypeStruct((B, C, HW), jnp.float32),
        compiler_params=params,
        name="psf_blend",
    )(S, D, HH, smin, sinv)

    return fused.reshape(B, C, H, W)


# speedup vs baseline: 3.8402x; 1.0462x over previous
import jax
import jax.numpy as jnp
from jax.experimental import pallas as pl
from jax.experimental.pallas import tpu as pltpu

EPS = 1e-8
HB = 16
LB = HB * 128
HALO = 512
R = LB // HALO


def _colsum7(x):
    w0 = x.shape[1]
    c2 = x[:, 128:w0 - 128] + x[:, 256:w0]
    c4 = c2[:, :w0 - 512] + c2[:, 256:w0 - 256]
    return c4[:, :LB] + c2[:, 512:512 + LB] + x[:, 896:896 + LB]


def _hmasks():
    lane = jax.lax.broadcasted_iota(jnp.int32, (1, LB), 1) % 128
    m_l = [(lane < 128 - s).astype(jnp.float32) for s in (1, 2, 3)]
    m_r = [(lane >= s).astype(jnp.float32) for s in (1, 2, 3)]
    return m_l, m_r


def _rowsum7(y, m_l, m_r):
    acc = y
    zcol = jnp.zeros((y.shape[0], 3), y.dtype)
    for s in (1, 2, 3):
        zs = zcol[:, :s]
        ls = jnp.concatenate([y[:, s:], zs], axis=1)
        rs = jnp.concatenate([zs, y[:, :-s]], axis=1)
        acc = acc + ls * m_l[s - 1] + rs * m_r[s - 1]
    return acc


def _box_stats(asm, m_l, m_r):
    s = _rowsum7(_colsum7(asm), m_l, m_r)
    s2 = _rowsum7(_colsum7(asm * asm), m_l, m_r)
    mean = s * (1.0 / 49.0)
    var = jnp.maximum((s2 - 49.0 * mean * mean) * (1.0 / 48.0), 0.0)
    return mean, var


def _assemble(center, top, bot, i, nt):
    t = jnp.where(i == 0, 0.0, top[0])
    b = jnp.where(i == nt - 1, 0.0, bot[0])
    return jnp.concatenate([t, center[0], b], axis=1)


def _main_body(fc, ft, fb, bc, bt, bb,
               we_r, be_r, wf_r, bf_r, wg_r, bg_r, wh_r, bh_r,
               s_ref, mn_ref, mx_ref, d_ref, hh_ref):
    i = pl.program_id(1)
    nt = pl.num_programs(1)

    m_l, m_r = _hmasks()
    basm = _assemble(bc, bt, bb, i, nt)
    bm, bv = _box_stats(basm, m_l, m_r)
    bstd = jnp.sqrt(bv)
    bn = (bc[0] - bm) / (bstd + EPS)
    GG = jnp.dot(wg_r[...], bn, preferred_element_type=jnp.float32) + bg_r[...]

    fasm = _assemble(fc, ft, fb, i, nt)
    fm, fv = _box_stats(fasm, m_l, m_r)
    fn = (fc[0] - fm) / (jnp.sqrt(fv) + EPS)
    FF = jnp.dot(wf_r[...], fn, preferred_element_type=jnp.float32) + bf_r[...]

    d = jnp.sum(FF * GG, axis=0, keepdims=True)
    nf = jnp.sqrt(jnp.sum(FF * FF, axis=0, keepdims=True))
    ng = jnp.sqrt(jnp.sum(GG * GG, axis=0, keepdims=True))
    S = d / (nf * ng)
    s_ref[0] = S
    mn = jnp.min(S)
    mx = jnp.max(S)

    @pl.when(i == 0)
    def _():
        mn_ref[...] = jnp.broadcast_to(mn, (1, 1, 128))
        mx_ref[...] = jnp.broadcast_to(mx, (1, 1, 128))

    @pl.when(i != 0)
    def _():
        mn_ref[...] = jnp.minimum(mn_ref[...], mn)
        mx_ref[...] = jnp.maximum(mx_ref[...], mx)

    ad = fn * bstd + bm
    EE = jnp.dot(we_r[...], ad, preferred_element_type=jnp.float32) + be_r[...]
    HH = jnp.dot(wh_r[...], bc[0], preferred_element_type=jnp.float32) + bh_r[...]
    hh_ref[0] = HH
    d_ref[0] = EE - HH


def _blend_body(s_in, d_in, hh_in, smin_r, sinv_r, o_ref):
    b = pl.program_id(0)
    sn = (s_in[0] - smin_r[b]) * sinv_r[b]
    o_ref[0] = hh_in[0] + sn * d_in[0]


def kernel(front, back, mask, we, be, wf, bf, wg, bg, wh, bh):
    del mask
    B, C, H, W = front.shape
    HW = H * W
    T = H // HB
    nblk = HW // HALO
    f2 = front.reshape(B, C, HW)
    b2 = back.reshape(B, C, HW)

    center = pl.BlockSpec((1, C, LB), lambda b, i: (b, 0, i))
    top = pl.BlockSpec((1, C, HALO),
                       lambda b, i: (b, 0, jnp.maximum(R * i - 1, 0)))
    bot = pl.BlockSpec((1, C, HALO),
                       lambda b, i: (b, 0, jnp.minimum(R * i + R, nblk - 1)))
    wspec = pl.BlockSpec((C, C), lambda b, i: (0, 0))
    bspec = pl.BlockSpec((C, 1), lambda b, i: (0, 0))
    srow = pl.BlockSpec((1, 1, LB), lambda b, i: (b, 0, i))
    params = pltpu.CompilerParams(
        dimension_semantics=("parallel", "arbitrary"),
        vmem_limit_bytes=56 * 1024 * 1024,
    )

    S, mn, mx, D, HH = pl.pallas_call(
        _main_body,
        grid=(B, T),
        in_specs=[center, top, bot, center, top, bot,
                  wspec, bspec, wspec, bspec, wspec, bspec, wspec, bspec],
        out_specs=[
            srow,
            pl.BlockSpec((1, 1, 128), lambda b, i: (b, 0, 0)),
            pl.BlockSpec((1, 1, 128), lambda b, i: (b, 0, 0)),
            center,
            center,
        ],
        out_shape=[
            jax.ShapeDtypeStruct((B, 1, HW), jnp.float32),
            jax.ShapeDtypeStruct((B, 1, 128), jnp.float32),
            jax.ShapeDtypeStruct((B, 1, 128), jnp.float32),
            jax.ShapeDtypeStruct((B, C, HW), jnp.float32),
            jax.ShapeDtypeStruct((B, C, HW), jnp.float32),
        ],
        compiler_params=params,
        name="psf_main",
    )(f2, f2, f2, b2, b2, b2,
      we, be.reshape(C, 1), wf, bf.reshape(C, 1),
      wg, bg.reshape(C, 1), wh, bh.reshape(C, 1))

    smin = mn[:, 0, 0]
    smax = mx[:, 0, 0]
    sinv = 1.0 / (smax - smin)

    fused = pl.pallas_call(
        _blend_body,
        grid=(B, T),
        in_specs=[srow, center, center,
                  pl.BlockSpec(memory_space=pltpu.SMEM),
                  pl.BlockSpec(memory_space=pltpu.SMEM)],
        out_specs=center,
        out_shape=jax.ShapeDtypeStruct((B, C, HW), jnp.float32),
        compiler_params=params,
        name="psf_blend",
    )(S, D, HH, smin, sinv)

    return fused.reshape(B, C, H, W)


# speedup vs baseline: 3.9122x; 1.0187x over previous
import jax
import jax.numpy as jnp
from jax.experimental import pallas as pl
from jax.experimental.pallas import tpu as pltpu

EPS = 1e-8
HB = 16
LB = HB * 128
HALO = 512
R = LB // HALO


def _colsum7(x):
    w0 = x.shape[1]
    c2 = x[:, 128:w0 - 128] + x[:, 256:w0]
    c4 = c2[:, :w0 - 512] + c2[:, 256:w0 - 256]
    return c4[:, :LB] + c2[:, 512:512 + LB] + x[:, 896:896 + LB]


def _hmasks():
    lane = jax.lax.broadcasted_iota(jnp.int32, (1, LB), 1) % 128
    m_l = [(lane < 128 - s).astype(jnp.float32) for s in (1, 2, 3)]
    m_r = [(lane >= s).astype(jnp.float32) for s in (1, 2, 3)]
    return m_l, m_r


def _rowsum7(y, m_l, m_r):
    acc = y
    zcol = jnp.zeros((y.shape[0], 3), y.dtype)
    for s in (1, 2, 3):
        zs = zcol[:, :s]
        ls = jnp.concatenate([y[:, s:], zs], axis=1)
        rs = jnp.concatenate([zs, y[:, :-s]], axis=1)
        acc = acc + ls * m_l[s - 1] + rs * m_r[s - 1]
    return acc


def _box_stats(asm, m_l, m_r):
    s = _rowsum7(_colsum7(asm), m_l, m_r)
    s2 = _rowsum7(_colsum7(asm * asm), m_l, m_r)
    mean = s * (1.0 / 49.0)
    var = jnp.maximum((s2 - 49.0 * mean * mean) * (1.0 / 48.0), 0.0)
    return mean, var


def _assemble(center, top, bot, i, nt):
    t = jnp.where(i == 0, 0.0, top[0])
    b = jnp.where(i == nt - 1, 0.0, bot[0])
    return jnp.concatenate([t, center[0], b], axis=1)


def _main_body(fc, ft, fb, bc, bt, bb,
               we_r, be_r, wf_r, bf_r, wg_r, bg_r, wh_r, bh_r,
               s_ref, mn_ref, mx_ref, d_ref, hh_ref):
    i = pl.program_id(1)
    nt = pl.num_programs(1)

    m_l, m_r = _hmasks()
    basm = _assemble(bc, bt, bb, i, nt)
    bm, bv = _box_stats(basm, m_l, m_r)
    bstd = jnp.sqrt(bv)
    bn = (bc[0] - bm) / (bstd + EPS)
    GG = jnp.dot(wg_r[...], bn, preferred_element_type=jnp.float32) + bg_r[...]

    fasm = _assemble(fc, ft, fb, i, nt)
    fm, fv = _box_stats(fasm, m_l, m_r)
    fn = (fc[0] - fm) / (jnp.sqrt(fv) + EPS)
    FF = jnp.dot(wf_r[...], fn, preferred_element_type=jnp.float32) + bf_r[...]

    d = jnp.sum(FF * GG, axis=0, keepdims=True)
    nf = jnp.sqrt(jnp.sum(FF * FF, axis=0, keepdims=True))
    ng = jnp.sqrt(jnp.sum(GG * GG, axis=0, keepdims=True))
    S = d / (nf * ng)
    s_ref[0] = S
    mn = jnp.min(S)
    mx = jnp.max(S)

    @pl.when(i == 0)
    def _():
        mn_ref[...] = jnp.broadcast_to(mn, (1, 1, 128))
        mx_ref[...] = jnp.broadcast_to(mx, (1, 1, 128))

    @pl.when(i != 0)
    def _():
        mn_ref[...] = jnp.minimum(mn_ref[...], mn)
        mx_ref[...] = jnp.maximum(mx_ref[...], mx)

    ad = fn * bstd + bm
    EE = jnp.dot(we_r[...], ad, preferred_element_type=jnp.float32) + be_r[...]
    HH = jnp.dot(wh_r[...], bc[0], preferred_element_type=jnp.float32) + bh_r[...]
    hh_ref[0] = HH.astype(jnp.bfloat16)
    d_ref[0] = (EE - HH).astype(jnp.bfloat16)


def _blend_body(s_in, d_in, hh_in, smin_r, sinv_r, o_ref):
    b = pl.program_id(0)
    sn = (s_in[0] - smin_r[b]) * sinv_r[b]
    o_ref[0] = hh_in[0].astype(jnp.float32) + sn * d_in[0].astype(jnp.float32)


def kernel(front, back, mask, we, be, wf, bf, wg, bg, wh, bh):
    del mask
    B, C, H, W = front.shape
    HW = H * W
    T = H // HB
    nblk = HW // HALO
    f2 = front.reshape(B, C, HW)
    b2 = back.reshape(B, C, HW)

    center = pl.BlockSpec((1, C, LB), lambda b, i: (b, 0, i))
    top = pl.BlockSpec((1, C, HALO),
                       lambda b, i: (b, 0, jnp.maximum(R * i - 1, 0)))
    bot = pl.BlockSpec((1, C, HALO),
                       lambda b, i: (b, 0, jnp.minimum(R * i + R, nblk - 1)))
    wspec = pl.BlockSpec((C, C), lambda b, i: (0, 0))
    bspec = pl.BlockSpec((C, 1), lambda b, i: (0, 0))
    srow = pl.BlockSpec((1, 1, LB), lambda b, i: (b, 0, i))
    params = pltpu.CompilerParams(
        dimension_semantics=("parallel", "arbitrary"),
        vmem_limit_bytes=56 * 1024 * 1024,
    )

    S, mn, mx, D, HH = pl.pallas_call(
        _main_body,
        grid=(B, T),
        in_specs=[center, top, bot, center, top, bot,
                  wspec, bspec, wspec, bspec, wspec, bspec, wspec, bspec],
        out_specs=[
            srow,
            pl.BlockSpec((1, 1, 128), lambda b, i: (b, 0, 0)),
            pl.BlockSpec((1, 1, 128), lambda b, i: (b, 0, 0)),
            center,
            center,
        ],
        out_shape=[
            jax.ShapeDtypeStruct((B, 1, HW), jnp.float32),
            jax.ShapeDtypeStruct((B, 1, 128), jnp.float32),
            jax.ShapeDtypeStruct((B, 1, 128), jnp.float32),
            jax.ShapeDtypeStruct((B, C, HW), jnp.bfloat16),
            jax.ShapeDtypeStruct((B, C, HW), jnp.bfloat16),
        ],
        compiler_params=params,
        name="psf_main",
    )(f2, f2, f2, b2, b2, b2,
      we, be.reshape(C, 1), wf, bf.reshape(C, 1),
      wg, bg.reshape(C, 1), wh, bh.reshape(C, 1))

    smin = mn[:, 0, 0]
    smax = mx[:, 0, 0]
    sinv = 1.0 / (smax - smin)

    fused = pl.pallas_call(
        _blend_body,
        grid=(B, T),
        in_specs=[srow, center, center,
                  pl.BlockSpec(memory_space=pltpu.SMEM),
                  pl.BlockSpec(memory_space=pltpu.SMEM)],
        out_specs=center,
        out_shape=jax.ShapeDtypeStruct((B, C, HW), jnp.float32),
        compiler_params=params,
        name="psf_blend",
    )(S, D, HH, smin, sinv)

    return fused.reshape(B, C, H, W)
